# baseline (speedup 1.0000x reference)
import functools

import jax
import jax.numpy as jnp
from jax import lax
from jax.experimental import pallas as pl
from jax.experimental.pallas import tpu as pltpu
from jax.experimental.pallas import tpu_sc as plsc

EPS = 1e-7
MAXN = 1.0 - 1e-5
G = 64
NC = 2
NS = 16


def _norm(x):
    return jnp.sqrt(jnp.sum(x * x, axis=-1, keepdims=True))


def _atanh(z):
    return 0.5 * jnp.log((1.0 + z) / (1.0 - z))


def _proj(x):
    n = jnp.clip(_norm(x), EPS, None)
    return jnp.where(n > MAXN, x / n * MAXN, x)


def _expmap0(u):
    n = jnp.clip(_norm(u), EPS, None)
    return jnp.tanh(n) * u / n


def _logmap0(p):
    n = _norm(p)
    nc = jnp.clip(n, EPS, MAXN)
    return _atanh(nc) * p / jnp.clip(n, EPS, None)


def _mobius_add(x, y):
    x2 = jnp.sum(x * x, axis=-1, keepdims=True)
    y2 = jnp.sum(y * y, axis=-1, keepdims=True)
    xy = jnp.sum(x * y, axis=-1, keepdims=True)
    num = (1.0 + 2.0 * xy + y2) * x + (1.0 - x2) * y
    den = 1.0 + 2.0 * xy + x2 * y2
    return num / jnp.clip(den, EPS, None)


def _linear_to_xt(h, W, b):
    t = _logmap0(h)
    u = lax.dot_general(t, W, (((1,), (1,)), ((), ())),
                        preferred_element_type=jnp.float32)
    hh = _proj(_expmap0(u))
    hb = _expmap0(b)
    hl = _proj(_mobius_add(hh, hb))
    return _logmap0(hl)


def _finish_agg(a0, a1, d0, d1):
    deg = jnp.clip(d0[:, :1] + d1[:, :1], 1.0, None)
    agg = (a0 + a1) / deg
    return _proj(_expmap0(agg))


def _t0_body(x_ref, w_ref, b_ref, o_ref):
    h = _proj(_expmap0(x_ref[...]))
    o_ref[...] = _linear_to_xt(h, w_ref[...], b_ref[...])


def _mid_body(a0_ref, a1_ref, d0_ref, d1_ref, w_ref, b_ref, o_ref):
    h = _finish_agg(a0_ref[...], a1_ref[...], d0_ref[...], d1_ref[...])
    t = _logmap0(h)
    t = jnp.where(t >= 0.0, t, 0.2 * t)
    h = _proj(_expmap0(t))
    o_ref[...] = _linear_to_xt(h, w_ref[...], b_ref[...])


def _final_body(nb, a0_ref, a1_ref, d0_ref, d1_ref, batch_ref, w_ref, b_ref,
                o_ref, pooled_s, cnt_s):
    i = pl.program_id(0)

    @pl.when(i == 0)
    def _init():
        pooled_s[...] = jnp.zeros_like(pooled_s)
        cnt_s[...] = jnp.zeros_like(cnt_s)

    h = _finish_agg(a0_ref[...], a1_ref[...], d0_ref[...], d1_ref[...])
    t = _logmap0(h)
    bids = batch_ref[0, 0, :]
    gids = lax.broadcasted_iota(jnp.int32, (G, 1), 0)
    m = (bids[None, :] == gids).astype(jnp.float32)
    pooled_s[...] += lax.dot_general(m, t, (((1,), (0,)), ((), ())),
                                     preferred_element_type=jnp.float32)
    cnt_s[...] += jnp.sum(m, axis=1, keepdims=True)

    @pl.when(i == nb - 1)
    def _finalize():
        pooled = pooled_s[...] / jnp.clip(cnt_s[...], 1.0, None)
        z = _expmap0(pooled)
        t4 = _logmap0(z)
        u = lax.dot_general(t4, w_ref[...], (((1,), (1,)), ((), ())),
                            preferred_element_type=jnp.float32)
        hh = _proj(_expmap0(u))
        hb = _expmap0(b_ref[...])
        o_ref[...] = _proj(_mobius_add(hh, hb))


def _tile_rows(n):
    rpt = (n // NS) // 8 * 8
    tail = n - NS * rpt
    return rpt, tail


def _make_agg_kernel(n, d, k, c):
    rpt, tail = _tile_rows(n)
    mesh = plsc.VectorSubcoreMesh(core_axis_name="c", subcore_axis_name="s")

    assert c % 8 == 0
    def body(xt_hbm, src_hbm, dst_hbm, zrow_hbm, acc_out,
             src_v, dst_v, rowbuf0, rowbuf1, acc_sh,
             gsem0, gsem1, ssem0, ssem1):
        cid = lax.axis_index("c")
        sid = lax.axis_index("s")
        wid = cid * NS + sid
        r0 = sid * rpt
        pltpu.sync_copy(zrow_hbm.at[pl.ds(r0, rpt)], acc_sh.at[pl.ds(r0, rpt)])
        if tail:
            @pl.when(sid == 0)
            def _zero_tail():
                t0 = NS * rpt
                pltpu.sync_copy(zrow_hbm.at[pl.ds(t0, tail)],
                                acc_sh.at[pl.ds(t0, tail)])
        pltpu.sync_copy(src_hbm.at[wid], src_v)
        pltpu.sync_copy(dst_hbm.at[wid], dst_v)
        plsc.subcore_barrier()

        def src_slice(j):
            return src_v.at[pl.ds(pl.multiple_of(j * c, 8), c)]

        def gstart(j, buf, sem):
            pltpu.async_copy(xt_hbm.at[src_slice(j)], buf, sem)

        def gwait(j, buf, sem):
            pltpu.make_async_copy(xt_hbm.at[src_slice(j)], buf, sem).wait()

        def finish(j, buf, sem):
            gwait(j, buf, sem)
            pltpu.sync_copy(buf, acc_sh.at[dst_v.at[j]], add=True)

        gstart(0, rowbuf0, gsem0)
        gstart(1, rowbuf1, gsem1)

        def step(jj, carry):
            j0 = 2 * jj
            finish(j0, rowbuf0, gsem0)

            @pl.when(j0 + 2 < k)
            def _p0():
                gstart(j0 + 2, rowbuf0, gsem0)

            finish(j0 + 1, rowbuf1, gsem1)

            @pl.when(j0 + 3 < k)
            def _p1():
                gstart(j0 + 3, rowbuf1, gsem1)

            return carry

        lax.fori_loop(0, k // 2, step, 0)
        if k % 2:
            finish(k - 1, rowbuf0, gsem0)
        plsc.subcore_barrier()
        pltpu.sync_copy(acc_sh.at[pl.ds(r0, rpt)],
                        acc_out.at[cid, pl.ds(r0, rpt)])
        if tail:
            @pl.when(sid == 0)
            def _export_tail():
                t0 = NS * rpt
                pltpu.sync_copy(acc_sh.at[pl.ds(t0, tail)],
                                acc_out.at[cid, pl.ds(t0, tail)])

    return pl.kernel(
        body,
        out_type=jax.ShapeDtypeStruct((NC, n, d), jnp.float32),
        mesh=mesh,
        scratch_types=[
            pltpu.VMEM((k * c,), jnp.int32),
            pltpu.VMEM((k, c), jnp.int32),
            pltpu.VMEM((c, d), jnp.float32),
            pltpu.VMEM((c, d), jnp.float32),
            pltpu.VMEM_SHARED((n + 8, d), jnp.float32),
            pltpu.SemaphoreType.DMA,
            pltpu.SemaphoreType.DMA,
            pltpu.SemaphoreType.DMA,
            pltpu.SemaphoreType.DMA,
        ],
    )


def _make_deg_kernel(n, d, k, c):
    rpt, tail = _tile_rows(n)
    mesh = plsc.VectorSubcoreMesh(core_axis_name="c", subcore_axis_name="s")

    def body(dst_hbm, zdeg_hbm, ones_hbm, deg_out,
             dst_v, ones_v, deg_sh, sem):
        cid = lax.axis_index("c")
        sid = lax.axis_index("s")
        wid = cid * NS + sid
        r0 = sid * rpt
        pltpu.sync_copy(zdeg_hbm.at[pl.ds(r0, rpt)], deg_sh.at[pl.ds(r0, rpt)])
        if tail:
            @pl.when(sid == 0)
            def _zero_tail():
                t0 = NS * rpt
                pltpu.sync_copy(zdeg_hbm.at[pl.ds(t0, tail)],
                                deg_sh.at[pl.ds(t0, tail)])
        pltpu.sync_copy(ones_hbm, ones_v)
        pltpu.sync_copy(dst_hbm.at[wid], dst_v)
        plsc.subcore_barrier()

        gg = 5

        def step(g, carry):
            base = g * gg
            for t in range(gg):
                pltpu.async_copy(ones_v, deg_sh.at[dst_v.at[base + t]],
                                 sem, add=True)
            for t in range(gg):
                pltpu.make_async_copy(ones_v, deg_sh.at[dst_v.at[base + t]],
                                      sem).wait()
            return carry

        lax.fori_loop(0, k // gg, step, 0)
        for t in range(k % gg):
            pltpu.sync_copy(ones_v, deg_sh.at[dst_v.at[(k // gg) * gg + t]],
                            add=True)
        plsc.subcore_barrier()
        pltpu.sync_copy(deg_sh.at[pl.ds(r0, rpt)],
                        deg_out.at[cid, pl.ds(r0, rpt)])
        if tail:
            @pl.when(sid == 0)
            def _export_tail():
                t0 = NS * rpt
                pltpu.sync_copy(deg_sh.at[pl.ds(t0, tail)],
                                deg_out.at[cid, pl.ds(t0, tail)])

    return pl.kernel(
        body,
        out_type=jax.ShapeDtypeStruct((NC, n, d), jnp.float32),
        mesh=mesh,
        scratch_types=[
            pltpu.VMEM((k, c), jnp.int32),
            pltpu.VMEM((c, d), jnp.float32),
            pltpu.VMEM_SHARED((n + 8, d), jnp.float32),
            pltpu.SemaphoreType.DMA,
        ],
    )


def _row_spec(bn, d):
    return pl.BlockSpec((bn, d), lambda i: (i, 0))


def _full_spec(shape):
    nd = len(shape)
    return pl.BlockSpec(shape, lambda i: (0,) * nd)


def _t0_call(x, w, b, bn):
    n, d = x.shape
    nb = n // bn
    return pl.pallas_call(
        _t0_body,
        grid=(nb,),
        in_specs=[_row_spec(bn, d), _full_spec(w.shape), _full_spec(b.shape)],
        out_specs=_row_spec(bn, d),
        out_shape=jax.ShapeDtypeStruct((n, d), jnp.float32),
    )(x, w, b)


def _mid_call(a0, a1, d0, d1, w, b, bn):
    n, d = a0.shape
    nb = n // bn
    return pl.pallas_call(
        _mid_body,
        grid=(nb,),
        in_specs=[_row_spec(bn, d), _row_spec(bn, d),
                  _row_spec(bn, d), _row_spec(bn, d),
                  _full_spec(w.shape), _full_spec(b.shape)],
        out_specs=_row_spec(bn, d),
        out_shape=jax.ShapeDtypeStruct((n, d), jnp.float32),
    )(a0, a1, d0, d1, w, b)


def _final_call(a0, a1, d0, d1, batch3, w, b, bn):
    n, d = a0.shape
    nb = n // bn
    return pl.pallas_call(
        functools.partial(_final_body, nb),
        grid=(nb,),
        in_specs=[_row_spec(bn, d), _row_spec(bn, d),
                  _row_spec(bn, d), _row_spec(bn, d),
                  pl.BlockSpec((1, 1, bn), lambda i: (i, 0, 0)),
                  _full_spec(w.shape), _full_spec(b.shape)],
        out_specs=_full_spec((G, d)),
        out_shape=jax.ShapeDtypeStruct((G, d), jnp.float32),
        scratch_shapes=[pltpu.VMEM((G, d), jnp.float32),
                        pltpu.VMEM((G, 1), jnp.float32)],
    )(a0, a1, d0, d1, batch3, w, b)


def kernel(x, edge_index, batch, W1, b1, W2, b2, W3, b3, W4, b4):
    n, d = x.shape
    e = edge_index.shape[1]
    nw = NC * NS
    ept = e // nw
    c = 96
    k = -(-ept // c)
    ecap = k * c
    bn = 1000
    nb = n // bn

    if ecap > ept:
        pad = jnp.zeros((nw, ecap - ept), jnp.int32)
        src3 = jnp.concatenate([edge_index[0].reshape(nw, ept), pad], axis=1)
        dst3 = jnp.concatenate(
            [edge_index[1].reshape(nw, ept), pad + n],
            axis=1).reshape(nw, k, c)
    else:
        src3 = edge_index[0].reshape(nw, ecap)
        dst3 = edge_index[1].reshape(nw, k, c)
    batch3 = batch.reshape(nb, 1, bn)
    zrow = jnp.zeros((n, d), jnp.float32)
    ones = jnp.ones((c, d), jnp.float32)
    b1r, b2r, b3r, b4r = (bb.reshape(1, -1) for bb in (b1, b2, b3, b4))

    agg = _make_agg_kernel(n, d, k, c)
    degk = _make_deg_kernel(n, d, k, c)

    deg = degk(dst3, zrow, ones)
    d0, d1 = deg[0], deg[1]
    xt = _t0_call(x, W1, b1r, bn)
    acc = agg(xt, src3, dst3, zrow)
    xt = _mid_call(acc[0], acc[1], d0, d1, W2, b2r, bn)
    acc = agg(xt, src3, dst3, zrow)
    xt = _mid_call(acc[0], acc[1], d0, d1, W3, b3r, bn)
    acc = agg(xt, src3, dst3, zrow)
    return _final_call(acc[0], acc[1], d0, d1, batch3, W4, b4r, bn)


# speedup vs baseline: 1.4370x; 1.4370x over previous
import functools

import jax
import jax.numpy as jnp
from jax import lax
from jax.experimental import pallas as pl
from jax.experimental.pallas import tpu as pltpu
from jax.experimental.pallas import tpu_sc as plsc

EPS = 1e-7
MAXN = 1.0 - 1e-5
G = 64
NC = 2
NS = 16


def _norm(x):
    return jnp.sqrt(jnp.sum(x * x, axis=-1, keepdims=True))


def _atanh(z):
    return 0.5 * jnp.log((1.0 + z) / (1.0 - z))


def _proj(x):
    n = jnp.clip(_norm(x), EPS, None)
    return jnp.where(n > MAXN, x / n * MAXN, x)


def _expmap0(u):
    n = jnp.clip(_norm(u), EPS, None)
    return jnp.tanh(n) * u / n


def _logmap0(p):
    n = _norm(p)
    nc = jnp.clip(n, EPS, MAXN)
    return _atanh(nc) * p / jnp.clip(n, EPS, None)


def _mobius_add(x, y):
    x2 = jnp.sum(x * x, axis=-1, keepdims=True)
    y2 = jnp.sum(y * y, axis=-1, keepdims=True)
    xy = jnp.sum(x * y, axis=-1, keepdims=True)
    num = (1.0 + 2.0 * xy + y2) * x + (1.0 - x2) * y
    den = 1.0 + 2.0 * xy + x2 * y2
    return num / jnp.clip(den, EPS, None)


def _linear_to_xt(h, W, b):
    t = _logmap0(h)
    u = lax.dot_general(t, W, (((1,), (1,)), ((), ())),
                        preferred_element_type=jnp.float32)
    hh = _proj(_expmap0(u))
    hb = _expmap0(b)
    hl = _proj(_mobius_add(hh, hb))
    return _logmap0(hl)


def _finish_agg(a0, a1, d0, d1):
    deg = jnp.clip(d0[:, :1] + d1[:, :1], 1.0, None)
    agg = (a0 + a1) / deg
    return _proj(_expmap0(agg))


def _t0_body(x_ref, w_ref, b_ref, o_ref):
    h = _proj(_expmap0(x_ref[...]))
    o_ref[...] = _linear_to_xt(h, w_ref[...], b_ref[...])


def _mid_body(a0_ref, a1_ref, d0_ref, d1_ref, w_ref, b_ref, o_ref):
    h = _finish_agg(a0_ref[...], a1_ref[...], d0_ref[...], d1_ref[...])
    t = _logmap0(h)
    t = jnp.where(t >= 0.0, t, 0.2 * t)
    h = _proj(_expmap0(t))
    o_ref[...] = _linear_to_xt(h, w_ref[...], b_ref[...])


def _final_body(nb, a0_ref, a1_ref, d0_ref, d1_ref, batch_ref, w_ref, b_ref,
                o_ref, pooled_s, cnt_s):
    i = pl.program_id(0)

    @pl.when(i == 0)
    def _init():
        pooled_s[...] = jnp.zeros_like(pooled_s)
        cnt_s[...] = jnp.zeros_like(cnt_s)

    h = _finish_agg(a0_ref[...], a1_ref[...], d0_ref[...], d1_ref[...])
    t = _logmap0(h)
    bids = batch_ref[0, 0, :]
    gids = lax.broadcasted_iota(jnp.int32, (G, 1), 0)
    m = (bids[None, :] == gids).astype(jnp.float32)
    pooled_s[...] += lax.dot_general(m, t, (((1,), (0,)), ((), ())),
                                     preferred_element_type=jnp.float32)
    cnt_s[...] += jnp.sum(m, axis=1, keepdims=True)

    @pl.when(i == nb - 1)
    def _finalize():
        pooled = pooled_s[...] / jnp.clip(cnt_s[...], 1.0, None)
        z = _expmap0(pooled)
        t4 = _logmap0(z)
        u = lax.dot_general(t4, w_ref[...], (((1,), (1,)), ((), ())),
                            preferred_element_type=jnp.float32)
        hh = _proj(_expmap0(u))
        hb = _expmap0(b_ref[...])
        o_ref[...] = _proj(_mobius_add(hh, hb))


def _tile_rows(n):
    rpt = (n // NS) // 8 * 8
    tail = n - NS * rpt
    return rpt, tail


def _make_agg_kernel(n, d, k, c):
    rpt, tail = _tile_rows(n)
    mesh = plsc.VectorSubcoreMesh(core_axis_name="c", subcore_axis_name="s")

    assert c % 8 == 0
    def body(xt_hbm, src_hbm, dst_hbm, zrow_hbm, acc_out,
             src_v, dst_v, rowbuf0, rowbuf1, acc_sh,
             gsem0, gsem1, ssem0, ssem1):
        cid = lax.axis_index("c")
        sid = lax.axis_index("s")
        wid = cid * NS + sid
        r0 = sid * rpt
        pltpu.sync_copy(zrow_hbm.at[pl.ds(r0, rpt)], acc_sh.at[pl.ds(r0, rpt)])
        if tail:
            @pl.when(sid == 0)
            def _zero_tail():
                t0 = NS * rpt
                pltpu.sync_copy(zrow_hbm.at[pl.ds(t0, tail)],
                                acc_sh.at[pl.ds(t0, tail)])
        pltpu.sync_copy(src_hbm.at[wid], src_v)
        pltpu.sync_copy(dst_hbm.at[wid], dst_v)
        plsc.subcore_barrier()

        def src_slice(j):
            return src_v.at[pl.ds(pl.multiple_of(j * c, 8), c)]

        def gstart(j, buf, sem):
            pltpu.async_copy(xt_hbm.at[src_slice(j)], buf, sem)

        def gwait(j, buf, sem):
            pltpu.make_async_copy(xt_hbm.at[src_slice(j)], buf, sem).wait()

        def finish(j, buf, sem):
            gwait(j, buf, sem)
            pltpu.sync_copy(buf, acc_sh.at[dst_v.at[j]], add=True)

        gstart(0, rowbuf0, gsem0)
        gstart(1, rowbuf1, gsem1)

        def step(jj, carry):
            j0 = 2 * jj
            finish(j0, rowbuf0, gsem0)

            @pl.when(j0 + 2 < k)
            def _p0():
                gstart(j0 + 2, rowbuf0, gsem0)

            finish(j0 + 1, rowbuf1, gsem1)

            @pl.when(j0 + 3 < k)
            def _p1():
                gstart(j0 + 3, rowbuf1, gsem1)

            return carry

        lax.fori_loop(0, k // 2, step, 0)
        if k % 2:
            finish(k - 1, rowbuf0, gsem0)
        plsc.subcore_barrier()
        pltpu.sync_copy(acc_sh.at[pl.ds(r0, rpt)],
                        acc_out.at[cid, pl.ds(r0, rpt)])
        if tail:
            @pl.when(sid == 0)
            def _export_tail():
                t0 = NS * rpt
                pltpu.sync_copy(acc_sh.at[pl.ds(t0, tail)],
                                acc_out.at[cid, pl.ds(t0, tail)])

    return pl.kernel(
        body,
        out_type=jax.ShapeDtypeStruct((NC, n, d), jnp.float32),
        mesh=mesh,
        scratch_types=[
            pltpu.VMEM((k * c,), jnp.int32),
            pltpu.VMEM((k, c), jnp.int32),
            pltpu.VMEM((c, d), jnp.float32),
            pltpu.VMEM((c, d), jnp.float32),
            pltpu.VMEM_SHARED((n + 8, d), jnp.float32),
            pltpu.SemaphoreType.DMA,
            pltpu.SemaphoreType.DMA,
            pltpu.SemaphoreType.DMA,
            pltpu.SemaphoreType.DMA,
        ],
    )


def _make_deg_kernel(n, d, k, c):
    rpt, tail = _tile_rows(n)
    mesh = plsc.VectorSubcoreMesh(core_axis_name="c", subcore_axis_name="s")

    def body(dst_hbm, zdeg_hbm, ones_hbm, deg_out,
             dst_v, ones_v, deg_sh, sem):
        cid = lax.axis_index("c")
        sid = lax.axis_index("s")
        wid = cid * NS + sid
        r0 = sid * rpt
        pltpu.sync_copy(zdeg_hbm.at[pl.ds(r0, rpt)], deg_sh.at[pl.ds(r0, rpt)])
        if tail:
            @pl.when(sid == 0)
            def _zero_tail():
                t0 = NS * rpt
                pltpu.sync_copy(zdeg_hbm.at[pl.ds(t0, tail)],
                                deg_sh.at[pl.ds(t0, tail)])
        pltpu.sync_copy(ones_hbm, ones_v)
        pltpu.sync_copy(dst_hbm.at[wid], dst_v)
        plsc.subcore_barrier()

        gg = 5

        def step(g, carry):
            base = g * gg
            for t in range(gg):
                pltpu.async_copy(ones_v, deg_sh.at[dst_v.at[base + t]],
                                 sem, add=True)
            for t in range(gg):
                pltpu.make_async_copy(ones_v, deg_sh.at[dst_v.at[base + t]],
                                      sem).wait()
            return carry

        lax.fori_loop(0, k // gg, step, 0)
        for t in range(k % gg):
            pltpu.sync_copy(ones_v, deg_sh.at[dst_v.at[(k // gg) * gg + t]],
                            add=True)
        plsc.subcore_barrier()
        pltpu.sync_copy(deg_sh.at[pl.ds(r0, rpt)],
                        deg_out.at[cid, pl.ds(r0, rpt)])
        if tail:
            @pl.when(sid == 0)
            def _export_tail():
                t0 = NS * rpt
                pltpu.sync_copy(deg_sh.at[pl.ds(t0, tail)],
                                deg_out.at[cid, pl.ds(t0, tail)])

    return pl.kernel(
        body,
        out_type=jax.ShapeDtypeStruct((NC, n, d), jnp.float32),
        mesh=mesh,
        scratch_types=[
            pltpu.VMEM((k, c), jnp.int32),
            pltpu.VMEM((c, d), jnp.float32),
            pltpu.VMEM_SHARED((n + 8, d), jnp.float32),
            pltpu.SemaphoreType.DMA,
        ],
    )


def _row_spec(bn, d):
    return pl.BlockSpec((bn, d), lambda i: (i, 0))


def _full_spec(shape):
    nd = len(shape)
    return pl.BlockSpec(shape, lambda i: (0,) * nd)


def _t0_call(x, w, b, bn):
    n, d = x.shape
    nb = n // bn
    return pl.pallas_call(
        _t0_body,
        grid=(nb,),
        in_specs=[_row_spec(bn, d), _full_spec(w.shape), _full_spec(b.shape)],
        out_specs=_row_spec(bn, d),
        out_shape=jax.ShapeDtypeStruct((n, d), jnp.float32),
    )(x, w, b)


def _mid_call(a0, a1, d0, d1, w, b, bn):
    n, d = a0.shape
    nb = n // bn
    return pl.pallas_call(
        _mid_body,
        grid=(nb,),
        in_specs=[_row_spec(bn, d), _row_spec(bn, d),
                  _row_spec(bn, 8), _row_spec(bn, 8),
                  _full_spec(w.shape), _full_spec(b.shape)],
        out_specs=_row_spec(bn, d),
        out_shape=jax.ShapeDtypeStruct((n, d), jnp.float32),
    )(a0, a1, d0, d1, w, b)


def _final_call(a0, a1, d0, d1, batch3, w, b, bn):
    n, d = a0.shape
    nb = n // bn
    return pl.pallas_call(
        functools.partial(_final_body, nb),
        grid=(nb,),
        in_specs=[_row_spec(bn, d), _row_spec(bn, d),
                  _row_spec(bn, 8), _row_spec(bn, 8),
                  pl.BlockSpec((1, 1, bn), lambda i: (i, 0, 0)),
                  _full_spec(w.shape), _full_spec(b.shape)],
        out_specs=_full_spec((G, d)),
        out_shape=jax.ShapeDtypeStruct((G, d), jnp.float32),
        scratch_shapes=[pltpu.VMEM((G, d), jnp.float32),
                        pltpu.VMEM((G, 1), jnp.float32)],
    )(a0, a1, d0, d1, batch3, w, b)


def kernel(x, edge_index, batch, W1, b1, W2, b2, W3, b3, W4, b4):
    n, d = x.shape
    e = edge_index.shape[1]
    nw = NC * NS
    ept = e // nw
    c = 80
    k = -(-ept // c)
    ecap = k * c
    bn = 1000
    nb = n // bn

    if ecap > ept:
        pad = jnp.zeros((nw, ecap - ept), jnp.int32)
        src3 = jnp.concatenate([edge_index[0].reshape(nw, ept), pad], axis=1)
        dst3 = jnp.concatenate(
            [edge_index[1].reshape(nw, ept), pad + n],
            axis=1).reshape(nw, k, c)
    else:
        src3 = edge_index[0].reshape(nw, ecap)
        dst3 = edge_index[1].reshape(nw, k, c)
    batch3 = batch.reshape(nb, 1, bn)
    zrow = jnp.zeros((n, d), jnp.float32)
    ones = jnp.ones((c, d), jnp.float32)
    b1r, b2r, b3r, b4r = (bb.reshape(1, -1) for bb in (b1, b2, b3, b4))

    agg = _make_agg_kernel(n, d, k, c)
    degk = _make_deg_kernel(n, d, k, c)

    deg = degk(dst3, zrow, ones)
    d0, d1 = deg[0, :, :8], deg[1, :, :8]
    xt = _t0_call(x, W1, b1r, bn)
    acc = agg(xt, src3, dst3, zrow)
    xt = _mid_call(acc[0], acc[1], d0, d1, W2, b2r, bn)
    acc = agg(xt, src3, dst3, zrow)
    xt = _mid_call(acc[0], acc[1], d0, d1, W3, b3r, bn)
    acc = agg(xt, src3, dst3, zrow)
    return _final_call(acc[0], acc[1], d0, d1, batch3, W4, b4r, bn)


# speedup vs baseline: 1.4386x; 1.0011x over previous
import functools

import jax
import jax.numpy as jnp
from jax import lax
from jax.experimental import pallas as pl
from jax.experimental.pallas import tpu as pltpu
from jax.experimental.pallas import tpu_sc as plsc

EPS = 1e-7
MAXN = 1.0 - 1e-5
G = 64
NC = 2
NS = 16


def _norm(x):
    return jnp.sqrt(jnp.sum(x * x, axis=-1, keepdims=True))


def _atanh(z):
    return 0.5 * jnp.log((1.0 + z) / (1.0 - z))


def _proj(x):
    n = jnp.clip(_norm(x), EPS, None)
    return jnp.where(n > MAXN, x / n * MAXN, x)


def _expmap0(u):
    n = jnp.clip(_norm(u), EPS, None)
    return jnp.tanh(n) * u / n


def _logmap0(p):
    n = _norm(p)
    nc = jnp.clip(n, EPS, MAXN)
    return _atanh(nc) * p / jnp.clip(n, EPS, None)


def _mobius_add(x, y):
    x2 = jnp.sum(x * x, axis=-1, keepdims=True)
    y2 = jnp.sum(y * y, axis=-1, keepdims=True)
    xy = jnp.sum(x * y, axis=-1, keepdims=True)
    num = (1.0 + 2.0 * xy + y2) * x + (1.0 - x2) * y
    den = 1.0 + 2.0 * xy + x2 * y2
    return num / jnp.clip(den, EPS, None)


def _linear_to_xt(h, W, b):
    t = _logmap0(h)
    u = lax.dot_general(t, W, (((1,), (1,)), ((), ())),
                        preferred_element_type=jnp.float32)
    hh = _proj(_expmap0(u))
    hb = _expmap0(b)
    hl = _proj(_mobius_add(hh, hb))
    return _logmap0(hl)


def _finish_agg(a0, a1, d0, d1):
    deg = jnp.clip(d0[:, :1] + d1[:, :1], 1.0, None)
    agg = (a0 + a1) / deg
    return _proj(_expmap0(agg))


def _t0_body(x_ref, w_ref, b_ref, o_ref):
    h = _proj(_expmap0(x_ref[...]))
    o_ref[...] = _linear_to_xt(h, w_ref[...], b_ref[...])


def _mid_body(a0_ref, a1_ref, d0_ref, d1_ref, w_ref, b_ref, o_ref):
    h = _finish_agg(a0_ref[...], a1_ref[...], d0_ref[...], d1_ref[...])
    t = _logmap0(h)
    t = jnp.where(t >= 0.0, t, 0.2 * t)
    h = _proj(_expmap0(t))
    o_ref[...] = _linear_to_xt(h, w_ref[...], b_ref[...])


def _final_body(nb, a0_ref, a1_ref, d0_ref, d1_ref, batch_ref, w_ref, b_ref,
                o_ref, pooled_s, cnt_s):
    i = pl.program_id(0)

    @pl.when(i == 0)
    def _init():
        pooled_s[...] = jnp.zeros_like(pooled_s)
        cnt_s[...] = jnp.zeros_like(cnt_s)

    h = _finish_agg(a0_ref[...], a1_ref[...], d0_ref[...], d1_ref[...])
    t = _logmap0(h)
    bids = batch_ref[0, 0, :]
    gids = lax.broadcasted_iota(jnp.int32, (G, 1), 0)
    m = (bids[None, :] == gids).astype(jnp.float32)
    pooled_s[...] += lax.dot_general(m, t, (((1,), (0,)), ((), ())),
                                     preferred_element_type=jnp.float32)
    cnt_s[...] += jnp.sum(m, axis=1, keepdims=True)

    @pl.when(i == nb - 1)
    def _finalize():
        pooled = pooled_s[...] / jnp.clip(cnt_s[...], 1.0, None)
        z = _expmap0(pooled)
        t4 = _logmap0(z)
        u = lax.dot_general(t4, w_ref[...], (((1,), (1,)), ((), ())),
                            preferred_element_type=jnp.float32)
        hh = _proj(_expmap0(u))
        hb = _expmap0(b_ref[...])
        o_ref[...] = _proj(_mobius_add(hh, hb))


def _tile_rows(n):
    rpt = (n // NS) // 8 * 8
    tail = n - NS * rpt
    return rpt, tail


def _make_agg_kernel(n, d, k, c):
    rpt, tail = _tile_rows(n)
    mesh = plsc.VectorSubcoreMesh(core_axis_name="c", subcore_axis_name="s")

    assert c % 8 == 0
    def body(xt_hbm, src_hbm, dst_hbm, zrow_hbm, acc_out,
             src_v, dst_v, rowbuf0, rowbuf1, acc_sh,
             gsem0, gsem1, ssem0, ssem1):
        cid = lax.axis_index("c")
        sid = lax.axis_index("s")
        wid = cid * NS + sid
        r0 = sid * rpt
        pltpu.sync_copy(zrow_hbm.at[pl.ds(r0, rpt)], acc_sh.at[pl.ds(r0, rpt)])
        if tail:
            @pl.when(sid == 0)
            def _zero_tail():
                t0 = NS * rpt
                pltpu.sync_copy(zrow_hbm.at[pl.ds(t0, tail)],
                                acc_sh.at[pl.ds(t0, tail)])
        pltpu.sync_copy(src_hbm.at[wid], src_v)
        pltpu.sync_copy(dst_hbm.at[wid], dst_v)
        plsc.subcore_barrier()

        def src_slice(j):
            return src_v.at[pl.ds(pl.multiple_of(j * c, 8), c)]

        def gstart(j, buf, sem):
            pltpu.async_copy(xt_hbm.at[src_slice(j)], buf, sem)

        def gwait(j, buf, sem):
            pltpu.make_async_copy(xt_hbm.at[src_slice(j)], buf, sem).wait()

        def finish(j, buf, sem):
            gwait(j, buf, sem)
            pltpu.sync_copy(buf, acc_sh.at[dst_v.at[j]], add=True)

        gstart(0, rowbuf0, gsem0)
        gstart(1, rowbuf1, gsem1)

        def step(jj, carry):
            j0 = 2 * jj
            finish(j0, rowbuf0, gsem0)

            @pl.when(j0 + 2 < k)
            def _p0():
                gstart(j0 + 2, rowbuf0, gsem0)

            finish(j0 + 1, rowbuf1, gsem1)

            @pl.when(j0 + 3 < k)
            def _p1():
                gstart(j0 + 3, rowbuf1, gsem1)

            return carry

        lax.fori_loop(0, k // 2, step, 0)
        if k % 2:
            finish(k - 1, rowbuf0, gsem0)
        plsc.subcore_barrier()
        pltpu.sync_copy(acc_sh.at[pl.ds(r0, rpt)],
                        acc_out.at[cid, pl.ds(r0, rpt)])
        if tail:
            @pl.when(sid == 0)
            def _export_tail():
                t0 = NS * rpt
                pltpu.sync_copy(acc_sh.at[pl.ds(t0, tail)],
                                acc_out.at[cid, pl.ds(t0, tail)])

    return pl.kernel(
        body,
        out_type=jax.ShapeDtypeStruct((NC, n, d), jnp.float32),
        mesh=mesh,
        scratch_types=[
            pltpu.VMEM((k * c,), jnp.int32),
            pltpu.VMEM((k, c), jnp.int32),
            pltpu.VMEM((c, d), jnp.float32),
            pltpu.VMEM((c, d), jnp.float32),
            pltpu.VMEM_SHARED((n + 8, d), jnp.float32),
            pltpu.SemaphoreType.DMA,
            pltpu.SemaphoreType.DMA,
            pltpu.SemaphoreType.DMA,
            pltpu.SemaphoreType.DMA,
        ],
    )


def _make_deg_kernel(n, d, k, c):
    rpt, tail = _tile_rows(n)
    mesh = plsc.VectorSubcoreMesh(core_axis_name="c", subcore_axis_name="s")

    def body(dst_hbm, zdeg_hbm, ones_hbm, deg_out,
             dst_v, ones_v, deg_sh, sem):
        cid = lax.axis_index("c")
        sid = lax.axis_index("s")
        wid = cid * NS + sid
        r0 = sid * rpt
        pltpu.sync_copy(zdeg_hbm.at[pl.ds(r0, rpt)], deg_sh.at[pl.ds(r0, rpt)])
        if tail:
            @pl.when(sid == 0)
            def _zero_tail():
                t0 = NS * rpt
                pltpu.sync_copy(zdeg_hbm.at[pl.ds(t0, tail)],
                                deg_sh.at[pl.ds(t0, tail)])
        pltpu.sync_copy(ones_hbm, ones_v)
        pltpu.sync_copy(dst_hbm.at[wid], dst_v)
        plsc.subcore_barrier()

        gg = 5

        def step(g, carry):
            base = g * gg
            for t in range(gg):
                pltpu.async_copy(ones_v, deg_sh.at[dst_v.at[base + t]],
                                 sem, add=True)
            for t in range(gg):
                pltpu.make_async_copy(ones_v, deg_sh.at[dst_v.at[base + t]],
                                      sem).wait()
            return carry

        lax.fori_loop(0, k // gg, step, 0)
        for t in range(k % gg):
            pltpu.sync_copy(ones_v, deg_sh.at[dst_v.at[(k // gg) * gg + t]],
                            add=True)
        plsc.subcore_barrier()
        pltpu.sync_copy(deg_sh.at[pl.ds(r0, rpt)],
                        deg_out.at[cid, pl.ds(r0, rpt)])
        if tail:
            @pl.when(sid == 0)
            def _export_tail():
                t0 = NS * rpt
                pltpu.sync_copy(deg_sh.at[pl.ds(t0, tail)],
                                deg_out.at[cid, pl.ds(t0, tail)])

    return pl.kernel(
        body,
        out_type=jax.ShapeDtypeStruct((NC, n, d), jnp.float32),
        mesh=mesh,
        scratch_types=[
            pltpu.VMEM((k, c), jnp.int32),
            pltpu.VMEM((c, d), jnp.float32),
            pltpu.VMEM_SHARED((n + 8, d), jnp.float32),
            pltpu.SemaphoreType.DMA,
        ],
    )


def _row_spec(bn, d):
    return pl.BlockSpec((bn, d), lambda i: (i, 0))


def _full_spec(shape):
    nd = len(shape)
    return pl.BlockSpec(shape, lambda i: (0,) * nd)


def _t0_call(x, w, b, bn):
    n, d = x.shape
    nb = n // bn
    return pl.pallas_call(
        _t0_body,
        grid=(nb,),
        in_specs=[_row_spec(bn, d), _full_spec(w.shape), _full_spec(b.shape)],
        out_specs=_row_spec(bn, d),
        out_shape=jax.ShapeDtypeStruct((n, d), jnp.float32),
    )(x, w, b)


def _mid_call(a0, a1, d0, d1, w, b, bn):
    n, d = a0.shape
    nb = n // bn
    return pl.pallas_call(
        _mid_body,
        grid=(nb,),
        in_specs=[_row_spec(bn, d), _row_spec(bn, d),
                  _row_spec(bn, 8), _row_spec(bn, 8),
                  _full_spec(w.shape), _full_spec(b.shape)],
        out_specs=_row_spec(bn, d),
        out_shape=jax.ShapeDtypeStruct((n, d), jnp.float32),
    )(a0, a1, d0, d1, w, b)


def _final_call(a0, a1, d0, d1, batch3, w, b, bn):
    n, d = a0.shape
    nb = n // bn
    return pl.pallas_call(
        functools.partial(_final_body, nb),
        grid=(nb,),
        in_specs=[_row_spec(bn, d), _row_spec(bn, d),
                  _row_spec(bn, 8), _row_spec(bn, 8),
                  pl.BlockSpec((1, 1, bn), lambda i: (i, 0, 0)),
                  _full_spec(w.shape), _full_spec(b.shape)],
        out_specs=_full_spec((G, d)),
        out_shape=jax.ShapeDtypeStruct((G, d), jnp.float32),
        scratch_shapes=[pltpu.VMEM((G, d), jnp.float32),
                        pltpu.VMEM((G, 1), jnp.float32)],
    )(a0, a1, d0, d1, batch3, w, b)


def kernel(x, edge_index, batch, W1, b1, W2, b2, W3, b3, W4, b4):
    n, d = x.shape
    e = edge_index.shape[1]
    nw = NC * NS
    ept = e // nw
    c = 80
    k = -(-ept // c)
    ecap = k * c
    bn = 1000
    nb = n // bn

    if ecap > ept:
        pad = jnp.zeros((nw, ecap - ept), jnp.int32)
        src3 = jnp.concatenate([edge_index[0].reshape(nw, ept), pad], axis=1)
        dst3 = jnp.concatenate(
            [edge_index[1].reshape(nw, ept), pad + n],
            axis=1).reshape(nw, k, c)
    else:
        src3 = edge_index[0].reshape(nw, ecap)
        dst3 = edge_index[1].reshape(nw, k, c)
    batch3 = batch.reshape(nb, 1, bn)
    zrow = jnp.zeros((n, d), jnp.float32)
    ones = jnp.ones((c, d), jnp.float32)
    b1r, b2r, b3r, b4r = (bb.reshape(1, -1) for bb in (b1, b2, b3, b4))

    agg = _make_agg_kernel(n, d, k, c)
    degk = _make_deg_kernel(n, d, k, c)

    xt = _t0_call(x, W1, b1r, bn)
    deg = degk(dst3, zrow, ones)
    d0, d1 = deg[0, :, :8], deg[1, :, :8]
    acc = agg(xt, src3, dst3, zrow)
    xt = _mid_call(acc[0], acc[1], d0, d1, W2, b2r, bn)
    acc = agg(xt, src3, dst3, zrow)
    xt = _mid_call(acc[0], acc[1], d0, d1, W3, b3r, bn)
    acc = agg(xt, src3, dst3, zrow)
    return _final_call(acc[0], acc[1], d0, d1, batch3, W4, b4r, bn)
